# Initial kernel scaffold; baseline (speedup 1.0000x reference)
import jax
import jax.numpy as jnp
from jax.experimental import pallas as pl


def kernel(node_attr, edge_attr, glob_attr, W1e, b1e, W2e, b2e, W1n, b1n, W2n, b2n, W1g, b1g, W2g, b2g, edge_index, batch):
    raise NotImplementedError("write your pallas kernel here")


# speedup vs baseline: 2.8396x; 2.8396x over previous
import functools

import jax
import jax.numpy as jnp
from jax import lax
from jax.experimental import pallas as pl
from jax.experimental.pallas import tpu as pltpu
from jax.experimental.pallas import tpu_sc as plsc

BN = 10000
BE = 320000
B = 16
ND = 128
ED = 16
GD = 16
EH = 2 * ND + ED + GD
NH = ND + ED + GD

NW = 32
C = 80
KPW = BE // (NW * C)
BNP = 10240

_SQRT_HALF = 0.7071067811865476


def _gelu(x):
    return 0.5 * x * (1.0 + lax.erf(x * _SQRT_HALF))


def _prep_body(na_ref, batch_ref, ga_ref, t_ref):
    na = na_ref[...]
    bids = batch_ref[0]
    onehot = (bids == lax.broadcasted_iota(jnp.int32, (1, B), 1)
              ).astype(jnp.float32)
    gpn = jnp.dot(onehot, ga_ref[...], preferred_element_type=jnp.float32)
    t_ref[...] = jnp.concatenate([na, gpn], axis=1)


def _prep_call(node_attr, batch3d, glob_attr):
    blk = 1000
    grid = BN // blk
    return pl.pallas_call(
        _prep_body,
        grid=(grid,),
        in_specs=[
            pl.BlockSpec((blk, ND), lambda i: (i, 0)),
            pl.BlockSpec((1, blk, 1), lambda i: (i, 0, 0)),
            pl.BlockSpec((B, GD), lambda i: (0, 0)),
        ],
        out_specs=pl.BlockSpec((blk, ND + GD), lambda i: (i, 0)),
        out_shape=jax.ShapeDtypeStruct((BN, ND + GD), jnp.float32),
    )(node_attr, batch3d, glob_attr)


@functools.cache
def _sc_mesh():
    return plsc.VectorSubcoreMesh(core_axis_name="c", subcore_axis_name="s")


@functools.cache
def _build_sc_gather():
    @functools.partial(
        pl.kernel,
        mesh=_sc_mesh(),
        out_type=(
            jax.ShapeDtypeStruct((BE, ND + GD), jnp.float32),
            jax.ShapeDtypeStruct((BE, ND), jnp.float32),
        ),
        scratch_types=[
            pltpu.VMEM((KPW, C), jnp.int32),
            pltpu.VMEM((KPW, C), jnp.int32),
            pltpu.VMEM((C, ND + GD), jnp.float32),
            pltpu.VMEM((C, ND), jnp.float32),
            pltpu.SemaphoreType.DMA,
            pltpu.SemaphoreType.DMA,
        ],
        compiler_params=pltpu.CompilerParams(use_tc_tiling_on_sc=False),
    )
    def _sc_gather(t_hbm, na_hbm, src_hbm, tgt_hbm, gs_hbm, gt_hbm,
                   src_v, tgt_v, buf_t, buf_n, sem_t, sem_n):
        cid = lax.axis_index("c")
        sid = lax.axis_index("s")
        wid = sid * 2 + cid
        pltpu.sync_copy(src_hbm.at[wid], src_v)
        pltpu.sync_copy(tgt_hbm.at[wid], tgt_v)

        def body(i, carry):
            cp_t = pltpu.async_copy(t_hbm.at[src_v.at[i]], buf_t, sem_t)
            cp_n = pltpu.async_copy(na_hbm.at[tgt_v.at[i]], buf_n, sem_n)
            cp_t.wait()
            cp_n.wait()
            base = (wid * KPW + i) * C
            pltpu.sync_copy(buf_t, gs_hbm.at[pl.ds(base, C)])
            pltpu.sync_copy(buf_n, gt_hbm.at[pl.ds(base, C)])
            return carry

        lax.fori_loop(0, KPW, body, 0)

    return _sc_gather


def _edge_body(gs_ref, gt_ref, ea_ref, w1s_ref, w1t_ref, w1c_ref, b1_ref,
               w2_ref, b2_ref, out_ref):
    h = jnp.dot(gs_ref[...], w1s_ref[...], preferred_element_type=jnp.float32)
    h = h + jnp.dot(gt_ref[...], w1t_ref[...], preferred_element_type=jnp.float32)
    h = h + jnp.dot(ea_ref[...], w1c_ref[...], preferred_element_type=jnp.float32)
    h = _gelu(h + b1_ref[...])
    o = jnp.dot(h, w2_ref[...], preferred_element_type=jnp.float32) + b2_ref[...]
    out_ref[...] = _gelu(o)


def _edge_call(gs, gt, edge_attr, w1s, w1t, w1c, b1e, w2e, b2e):
    blk = 1280
    grid = BE // blk
    full = lambda r, c: pl.BlockSpec((r, c), lambda i: (0, 0))
    return pl.pallas_call(
        _edge_body,
        grid=(grid,),
        in_specs=[
            pl.BlockSpec((blk, ND + GD), lambda i: (i, 0)),
            pl.BlockSpec((blk, ND), lambda i: (i, 0)),
            pl.BlockSpec((blk, ED), lambda i: (i, 0)),
            full(ND + GD, EH),
            full(ND, EH),
            full(ED, EH),
            full(1, EH),
            full(EH, ED),
            full(1, ED),
        ],
        out_specs=pl.BlockSpec((blk, ED), lambda i: (i, 0)),
        out_shape=jax.ShapeDtypeStruct((BE, ED), jnp.float32),
    )(gs, gt, edge_attr, w1s, w1t, w1c, b1e, w2e, b2e)


@functools.cache
def _build_sc_scatter():
    @functools.partial(
        pl.kernel,
        mesh=_sc_mesh(),
        out_type=jax.ShapeDtypeStruct((2, BNP, ED), jnp.float32),
        scratch_types=[
            pltpu.VMEM((KPW, C), jnp.int32),
            pltpu.VMEM((C, ED), jnp.float32),
            pltpu.VMEM((BNP // 16, ED), jnp.float32),
            pltpu.VMEM_SHARED((BNP, ED), jnp.float32),
        ],
        compiler_params=pltpu.CompilerParams(use_tc_tiling_on_sc=False),
    )
    def _sc_scatter(e_hbm, tgt_hbm, p_hbm, tgt_v, buf_e, zbuf, acc):
        cid = lax.axis_index("c")
        sid = lax.axis_index("s")
        wid = sid * 2 + cid
        rows = BNP // 16

        def zb(i, carry):
            zbuf[i, :] = jnp.zeros((ED,), jnp.float32)
            return carry

        lax.fori_loop(0, rows, zb, 0)
        pltpu.sync_copy(zbuf, acc.at[pl.ds(sid * rows, rows)])
        plsc.subcore_barrier()

        pltpu.sync_copy(tgt_hbm.at[wid], tgt_v)

        def body(i, carry):
            base = (wid * KPW + i) * C
            pltpu.sync_copy(e_hbm.at[pl.ds(base, C)], buf_e)
            pltpu.sync_copy(buf_e, acc.at[tgt_v.at[i]], add=True)
            return carry

        lax.fori_loop(0, KPW, body, 0)
        plsc.subcore_barrier()

        @pl.when(sid == 0)
        def _():
            pltpu.sync_copy(acc, p_hbm.at[cid])

    return _sc_scatter


def _node_body(t_ref, p0_ref, p1_ref, batch_ref, w1a_ref, w1b_ref, w1c_ref,
               b1_ref, w2_ref, b2_ref, ne_ref, n2g_ref, e2g_ref):
    i = pl.program_id(0)
    t = t_ref[...]
    na = t[:, :ND]
    gpn = t[:, ND:]
    e2n = p0_ref[...] + p1_ref[...]
    h = jnp.dot(na, w1a_ref[...], preferred_element_type=jnp.float32)
    h = h + jnp.dot(e2n, w1b_ref[...], preferred_element_type=jnp.float32)
    h = h + jnp.dot(gpn, w1c_ref[...], preferred_element_type=jnp.float32)
    h = _gelu(h + b1_ref[...])
    ne = _gelu(jnp.dot(h, w2_ref[...], preferred_element_type=jnp.float32)
               + b2_ref[...])
    ne_ref[...] = ne

    inf = jnp.float32(jnp.inf)

    @pl.when(i == 0)
    def _():
        n2g_ref[...] = jnp.full((B, ND), inf, jnp.float32)
        e2g_ref[...] = jnp.full((B, ED), inf, jnp.float32)

    bids = batch_ref[0]
    nmins = []
    emins = []
    for g in range(B):
        m = bids == g
        nmins.append(jnp.min(jnp.where(m, ne, inf), axis=0))
        emins.append(jnp.min(jnp.where(m, e2n, inf), axis=0))
    n2g_ref[...] = jnp.minimum(n2g_ref[...], jnp.stack(nmins))
    e2g_ref[...] = jnp.minimum(e2g_ref[...], jnp.stack(emins))


def _node_call(t, p0, p1, batch3d, w1a, w1b, w1c, b1n, w2n, b2n):
    blk = 1000
    grid = BN // blk
    full = lambda r, c: pl.BlockSpec((r, c), lambda i: (0, 0))
    return pl.pallas_call(
        _node_body,
        grid=(grid,),
        in_specs=[
            pl.BlockSpec((blk, ND + GD), lambda i: (i, 0)),
            pl.BlockSpec((blk, ED), lambda i: (i, 0)),
            pl.BlockSpec((blk, ED), lambda i: (i, 0)),
            pl.BlockSpec((1, blk, 1), lambda i: (i, 0, 0)),
            full(ND, NH),
            full(ED, NH),
            full(GD, NH),
            full(1, NH),
            full(NH, ND),
            full(1, ND),
        ],
        out_specs=[
            pl.BlockSpec((blk, ND), lambda i: (i, 0)),
            full(B, ND),
            full(B, ED),
        ],
        out_shape=[
            jax.ShapeDtypeStruct((BN, ND), jnp.float32),
            jax.ShapeDtypeStruct((B, ND), jnp.float32),
            jax.ShapeDtypeStruct((B, ED), jnp.float32),
        ],
    )(t, p0, p1, batch3d, w1a, w1b, w1c, b1n, w2n, b2n)


def _glob_body(n2g_ref, e2g_ref, ga_ref, w1a_ref, w1b_ref, w1c_ref, b1_ref,
               w2_ref, b2_ref, out_ref):
    h = jnp.dot(n2g_ref[...], w1a_ref[...], preferred_element_type=jnp.float32)
    h = h + jnp.dot(e2g_ref[...], w1b_ref[...], preferred_element_type=jnp.float32)
    h = h + jnp.dot(ga_ref[...], w1c_ref[...], preferred_element_type=jnp.float32)
    h = _gelu(h + b1_ref[...])
    o = jnp.dot(h, w2_ref[...], preferred_element_type=jnp.float32) + b2_ref[...]
    out_ref[...] = _gelu(o)


def _glob_call(n2g, e2g, glob_attr, w1a, w1b, w1c, b1g, w2g, b2g):
    return pl.pallas_call(
        _glob_body,
        out_shape=jax.ShapeDtypeStruct((B, GD), jnp.float32),
    )(n2g, e2g, glob_attr, w1a, w1b, w1c, b1g, w2g, b2g)


def _sc_gather_call(t, node_attr, src2d, tgt2d):
    return _build_sc_gather()(t, node_attr, src2d, tgt2d)


def _sc_scatter_call(edge_emb, tgt2d):
    return _build_sc_scatter()(edge_emb, tgt2d)


def kernel(node_attr, edge_attr, glob_attr,
           W1e, b1e, W2e, b2e,
           W1n, b1n, W2n, b2n,
           W1g, b1g, W2g, b2g,
           edge_index, batch):
    src2d = edge_index[0].reshape(NW, KPW, C)
    tgt2d = edge_index[1].reshape(NW, KPW, C)
    batch3d = batch.reshape(BN // 1000, 1000, 1)

    w1s = jnp.concatenate([W1e[0:ND], W1e[2 * ND + ED:]], axis=0)
    w1t = W1e[ND:2 * ND]
    w1c = W1e[2 * ND:2 * ND + ED]

    t = _prep_call(node_attr, batch3d, glob_attr)
    gs, gt = _sc_gather_call(t, node_attr, src2d, tgt2d)
    edge_emb = _edge_call(gs, gt, edge_attr, w1s, w1t, w1c,
                          b1e.reshape(1, EH), W2e, b2e.reshape(1, ED))
    p = _sc_scatter_call(edge_emb, tgt2d)
    node_emb, n2g, e2g = _node_call(
        t, p[0, :BN], p[1, :BN], batch3d,
        W1n[0:ND], W1n[ND:ND + ED], W1n[ND + ED:], b1n.reshape(1, NH),
        W2n, b2n.reshape(1, ND))
    glob_emb = _glob_call(
        n2g, e2g, glob_attr,
        W1g[0:ND], W1g[ND:ND + ED], W1g[ND + ED:], b1g.reshape(1, NH),
        W2g, b2g.reshape(1, GD))
    return node_emb, edge_emb, glob_emb


# speedup vs baseline: 2.9785x; 1.0489x over previous
import functools

import jax
import jax.numpy as jnp
from jax import lax
from jax.experimental import pallas as pl
from jax.experimental.pallas import tpu as pltpu
from jax.experimental.pallas import tpu_sc as plsc

BN = 10000
BE = 320000
B = 16
ND = 128
ED = 16
GD = 16
EH = 2 * ND + ED + GD
NH = ND + ED + GD

NW = 32
C = 80
KPW = BE // (NW * C)
NBUF = 5
EC = 2000
BNP = 10240

_SQRT_HALF = 0.7071067811865476


def _gelu(x):
    return 0.5 * x * (1.0 + lax.erf(x * _SQRT_HALF))


def _prep_body(na_ref, batch_ref, ga_ref, t_ref):
    na = na_ref[...]
    bids = batch_ref[0]
    onehot = (bids == lax.broadcasted_iota(jnp.int32, (1, B), 1)
              ).astype(jnp.float32)
    gpn = jnp.dot(onehot, ga_ref[...], preferred_element_type=jnp.float32)
    t_ref[...] = jnp.concatenate([na, gpn], axis=1)


def _prep_call(node_attr, batch3d, glob_attr):
    blk = 1000
    grid = BN // blk
    return pl.pallas_call(
        _prep_body,
        grid=(grid,),
        in_specs=[
            pl.BlockSpec((blk, ND), lambda i: (i, 0)),
            pl.BlockSpec((1, blk, 1), lambda i: (i, 0, 0)),
            pl.BlockSpec((B, GD), lambda i: (0, 0)),
        ],
        out_specs=pl.BlockSpec((blk, ND + GD), lambda i: (i, 0)),
        out_shape=jax.ShapeDtypeStruct((BN, ND + GD), jnp.float32),
    )(node_attr, batch3d, glob_attr)


@functools.cache
def _sc_mesh():
    return plsc.VectorSubcoreMesh(core_axis_name="c", subcore_axis_name="s")


@functools.cache
def _build_sc_gather():
    @functools.partial(
        pl.kernel,
        mesh=_sc_mesh(),
        out_type=(
            jax.ShapeDtypeStruct((BE, ND + GD), jnp.float32),
            jax.ShapeDtypeStruct((BE, ND), jnp.float32),
        ),
        scratch_types=[
            pltpu.VMEM((KPW, C), jnp.int32),
            pltpu.VMEM((KPW, C), jnp.int32),
            pltpu.VMEM((NBUF, C, ND + GD), jnp.float32),
            pltpu.VMEM((NBUF, C, ND), jnp.float32),
            pltpu.SemaphoreType.DMA((NBUF,)),
            pltpu.SemaphoreType.DMA((NBUF,)),
            pltpu.SemaphoreType.DMA((NBUF,)),
            pltpu.SemaphoreType.DMA((NBUF,)),
        ],
        compiler_params=pltpu.CompilerParams(use_tc_tiling_on_sc=False),
    )
    def _sc_gather(t_hbm, na_hbm, src_hbm, tgt_hbm, gs_hbm, gt_hbm,
                   src_v, tgt_v, buf_t, buf_n, sem_gt, sem_gn, sem_wt, sem_wn):
        cid = lax.axis_index("c")
        sid = lax.axis_index("s")
        wid = sid * 2 + cid
        pltpu.sync_copy(src_hbm.at[wid], src_v)
        pltpu.sync_copy(tgt_hbm.at[wid], tgt_v)

        def gather_start(i, b):
            pltpu.async_copy(t_hbm.at[src_v.at[i]], buf_t.at[b], sem_gt.at[b])
            pltpu.async_copy(na_hbm.at[tgt_v.at[i]], buf_n.at[b], sem_gn.at[b])

        def finish(j, b):
            pltpu.make_async_copy(t_hbm.at[src_v.at[j]], buf_t.at[b],
                                  sem_gt.at[b]).wait()
            pltpu.make_async_copy(na_hbm.at[tgt_v.at[j]], buf_n.at[b],
                                  sem_gn.at[b]).wait()
            base = (wid * KPW + j) * C
            pltpu.async_copy(buf_t.at[b], gs_hbm.at[pl.ds(base, C)],
                             sem_wt.at[b])
            pltpu.async_copy(buf_n.at[b], gt_hbm.at[pl.ds(base, C)],
                             sem_wn.at[b])

        def wb_wait(b):
            pltpu.make_async_copy(buf_t.at[b], gs_hbm.at[pl.ds(0, C)],
                                  sem_wt.at[b]).wait()
            pltpu.make_async_copy(buf_n.at[b], gt_hbm.at[pl.ds(0, C)],
                                  sem_wn.at[b]).wait()

        def round_(g, carry):
            for b in range(NBUF):
                i = g * NBUF + b

                @pl.when(i >= NBUF)
                def _():
                    wb_wait(b)

                gather_start(i, b)
                bp = (b - 1) % NBUF

                @pl.when(i >= 1)
                def _():
                    finish(i - 1, bp)

            return carry

        lax.fori_loop(0, KPW // NBUF, round_, 0)
        finish(KPW - 1, (KPW - 1) % NBUF)
        for b in range(NBUF):
            wb_wait(b)

    return _sc_gather


def _edge_body(gs_ref, gt_ref, ea_ref, w1s_ref, w1t_ref, w1c_ref, b1_ref,
               w2_ref, b2_ref, out_ref):
    h = jnp.dot(gs_ref[...], w1s_ref[...], preferred_element_type=jnp.float32)
    h = h + jnp.dot(gt_ref[...], w1t_ref[...], preferred_element_type=jnp.float32)
    h = h + jnp.dot(ea_ref[...], w1c_ref[...], preferred_element_type=jnp.float32)
    h = _gelu(h + b1_ref[...])
    o = jnp.dot(h, w2_ref[...], preferred_element_type=jnp.float32) + b2_ref[...]
    out_ref[...] = _gelu(o)


def _edge_call(gs, gt, edge_attr, w1s, w1t, w1c, b1e, w2e, b2e):
    blk = 1280
    grid = BE // blk
    full = lambda r, c: pl.BlockSpec((r, c), lambda i: (0, 0))
    return pl.pallas_call(
        _edge_body,
        grid=(grid,),
        in_specs=[
            pl.BlockSpec((blk, ND + GD), lambda i: (i, 0)),
            pl.BlockSpec((blk, ND), lambda i: (i, 0)),
            pl.BlockSpec((blk, ED), lambda i: (i, 0)),
            full(ND + GD, EH),
            full(ND, EH),
            full(ED, EH),
            full(1, EH),
            full(EH, ED),
            full(1, ED),
        ],
        out_specs=pl.BlockSpec((blk, ED), lambda i: (i, 0)),
        out_shape=jax.ShapeDtypeStruct((BE, ED), jnp.float32),
    )(gs, gt, edge_attr, w1s, w1t, w1c, b1e, w2e, b2e)


@functools.cache
def _build_sc_scatter():
    @functools.partial(
        pl.kernel,
        mesh=_sc_mesh(),
        out_type=jax.ShapeDtypeStruct((2, BNP, ED), jnp.float32),
        scratch_types=[
            pltpu.VMEM((KPW, C), jnp.int32),
            pltpu.VMEM((2, EC, ED), jnp.float32),
            pltpu.VMEM((BNP // 16, ED), jnp.float32),
            pltpu.VMEM_SHARED((BNP, ED), jnp.float32),
            pltpu.SemaphoreType.DMA((2,)),
            pltpu.SemaphoreType.DMA((2,)),
        ],
        compiler_params=pltpu.CompilerParams(use_tc_tiling_on_sc=False),
    )
    def _sc_scatter(e_hbm, tgt_hbm, p_hbm, tgt_v, buf_e, zbuf, acc,
                    sem_l, sem_s):
        cid = lax.axis_index("c")
        sid = lax.axis_index("s")
        wid = sid * 2 + cid
        rows = BNP // 16
        cpe = EC // C
        nsc = (KPW * C) // EC

        def zb(i, carry):
            zbuf[i, :] = jnp.zeros((ED,), jnp.float32)
            return carry

        lax.fori_loop(0, rows, zb, 0)
        pltpu.sync_copy(zbuf, acc.at[pl.ds(sid * rows, rows)])

        pltpu.sync_copy(tgt_hbm.at[wid], tgt_v)
        plsc.subcore_barrier()

        def load_start(s, b):
            base = wid * KPW * C + s * EC
            pltpu.async_copy(e_hbm.at[pl.ds(base, EC)], buf_e.at[b],
                             sem_l.at[b])

        def scat(s, b):
            pltpu.make_async_copy(e_hbm.at[pl.ds(0, EC)], buf_e.at[b],
                                  sem_l.at[b]).wait()

            def one(j, carry):
                idx = tgt_v.at[s * cpe + j]
                pltpu.async_copy(buf_e.at[b].at[pl.ds(j * C, C)],
                                 acc.at[idx], sem_s.at[b], add=True)
                return carry

            lax.fori_loop(0, cpe, one, 0)

        def drain(b):
            def one(j, carry):
                pltpu.make_async_copy(buf_e.at[b].at[pl.ds(0, C)],
                                      acc.at[tgt_v.at[0]], sem_s.at[b]).wait()
                return carry

            lax.fori_loop(0, cpe, one, 0)

        load_start(0, 0)

        def body(s, carry):
            b = s % 2
            bn = (s + 1) % 2

            @pl.when(s + 1 < nsc)
            def _():
                @pl.when(s >= 1)
                def _():
                    drain(bn)

                load_start(s + 1, bn)

            scat(s, b)
            return carry

        lax.fori_loop(0, nsc, body, 0)
        drain(0)
        drain(1)
        plsc.subcore_barrier()

        @pl.when(sid == 0)
        def _():
            pltpu.sync_copy(acc, p_hbm.at[cid])

    return _sc_scatter


def _node_body(t_ref, p0_ref, p1_ref, batch_ref, w1a_ref, w1b_ref, w1c_ref,
               b1_ref, w2_ref, b2_ref, ne_ref, n2g_ref, e2g_ref):
    i = pl.program_id(0)
    t = t_ref[...]
    na = t[:, :ND]
    gpn = t[:, ND:]
    e2n = p0_ref[...] + p1_ref[...]
    h = jnp.dot(na, w1a_ref[...], preferred_element_type=jnp.float32)
    h = h + jnp.dot(e2n, w1b_ref[...], preferred_element_type=jnp.float32)
    h = h + jnp.dot(gpn, w1c_ref[...], preferred_element_type=jnp.float32)
    h = _gelu(h + b1_ref[...])
    ne = _gelu(jnp.dot(h, w2_ref[...], preferred_element_type=jnp.float32)
               + b2_ref[...])
    ne_ref[...] = ne

    inf = jnp.float32(jnp.inf)

    @pl.when(i == 0)
    def _():
        n2g_ref[...] = jnp.full((B, ND), inf, jnp.float32)
        e2g_ref[...] = jnp.full((B, ED), inf, jnp.float32)

    bids = batch_ref[0]
    nmins = []
    emins = []
    for g in range(B):
        m = bids == g
        nmins.append(jnp.min(jnp.where(m, ne, inf), axis=0))
        emins.append(jnp.min(jnp.where(m, e2n, inf), axis=0))
    n2g_ref[...] = jnp.minimum(n2g_ref[...], jnp.stack(nmins))
    e2g_ref[...] = jnp.minimum(e2g_ref[...], jnp.stack(emins))


def _node_call(t, p0, p1, batch3d, w1a, w1b, w1c, b1n, w2n, b2n):
    blk = 1000
    grid = BN // blk
    full = lambda r, c: pl.BlockSpec((r, c), lambda i: (0, 0))
    return pl.pallas_call(
        _node_body,
        grid=(grid,),
        in_specs=[
            pl.BlockSpec((blk, ND + GD), lambda i: (i, 0)),
            pl.BlockSpec((blk, ED), lambda i: (i, 0)),
            pl.BlockSpec((blk, ED), lambda i: (i, 0)),
            pl.BlockSpec((1, blk, 1), lambda i: (i, 0, 0)),
            full(ND, NH),
            full(ED, NH),
            full(GD, NH),
            full(1, NH),
            full(NH, ND),
            full(1, ND),
        ],
        out_specs=[
            pl.BlockSpec((blk, ND), lambda i: (i, 0)),
            full(B, ND),
            full(B, ED),
        ],
        out_shape=[
            jax.ShapeDtypeStruct((BN, ND), jnp.float32),
            jax.ShapeDtypeStruct((B, ND), jnp.float32),
            jax.ShapeDtypeStruct((B, ED), jnp.float32),
        ],
    )(t, p0, p1, batch3d, w1a, w1b, w1c, b1n, w2n, b2n)


def _glob_body(n2g_ref, e2g_ref, ga_ref, w1a_ref, w1b_ref, w1c_ref, b1_ref,
               w2_ref, b2_ref, out_ref):
    h = jnp.dot(n2g_ref[...], w1a_ref[...], preferred_element_type=jnp.float32)
    h = h + jnp.dot(e2g_ref[...], w1b_ref[...], preferred_element_type=jnp.float32)
    h = h + jnp.dot(ga_ref[...], w1c_ref[...], preferred_element_type=jnp.float32)
    h = _gelu(h + b1_ref[...])
    o = jnp.dot(h, w2_ref[...], preferred_element_type=jnp.float32) + b2_ref[...]
    out_ref[...] = _gelu(o)


def _glob_call(n2g, e2g, glob_attr, w1a, w1b, w1c, b1g, w2g, b2g):
    return pl.pallas_call(
        _glob_body,
        out_shape=jax.ShapeDtypeStruct((B, GD), jnp.float32),
    )(n2g, e2g, glob_attr, w1a, w1b, w1c, b1g, w2g, b2g)


def _sc_gather_call(t, node_attr, src2d, tgt2d):
    return _build_sc_gather()(t, node_attr, src2d, tgt2d)


def _sc_scatter_call(edge_emb, tgt2d):
    return _build_sc_scatter()(edge_emb, tgt2d)


def kernel(node_attr, edge_attr, glob_attr,
           W1e, b1e, W2e, b2e,
           W1n, b1n, W2n, b2n,
           W1g, b1g, W2g, b2g,
           edge_index, batch):
    src2d = edge_index[0].reshape(NW, KPW, C)
    tgt2d = edge_index[1].reshape(NW, KPW, C)
    batch3d = batch.reshape(BN // 1000, 1000, 1)

    w1s = jnp.concatenate([W1e[0:ND], W1e[2 * ND + ED:]], axis=0)
    w1t = W1e[ND:2 * ND]
    w1c = W1e[2 * ND:2 * ND + ED]

    t = _prep_call(node_attr, batch3d, glob_attr)
    gs, gt = _sc_gather_call(t, node_attr, src2d, tgt2d)
    edge_emb = _edge_call(gs, gt, edge_attr, w1s, w1t, w1c,
                          b1e.reshape(1, EH), W2e, b2e.reshape(1, ED))
    p = _sc_scatter_call(edge_emb, tgt2d)
    node_emb, n2g, e2g = _node_call(
        t, p[0, :BN], p[1, :BN], batch3d,
        W1n[0:ND], W1n[ND:ND + ED], W1n[ND + ED:], b1n.reshape(1, NH),
        W2n, b2n.reshape(1, ND))
    glob_emb = _glob_call(
        n2g, e2g, glob_attr,
        W1g[0:ND], W1g[ND:ND + ED], W1g[ND + ED:], b1g.reshape(1, NH),
        W2g, b2g.reshape(1, GD))
    return node_emb, edge_emb, glob_emb
